# baseline (speedup 1.0000x reference)
import jax
import jax.numpy as jnp
from jax import lax
from jax.experimental import pallas as pl
from jax.experimental.pallas import tpu as pltpu
from jax.experimental.pallas import tpu_sc as plsc

NC = 2
NS = 16
NW = NC * NS
PAIRS = 16384
D = 32
RPS = 4
SROWS = 250000
PAIRS_PER_W = PAIRS // NW
ROWS_PER_W = 2 * PAIRS_PER_W
NCHUNK = 8
CHUNK = ROWS_PER_W // NCHUNK
CPAIRS = CHUNK // 2
CGROUPS = CPAIRS // 16


def _ip_body(emb_hbm, blk_hbm, sub_hbm, out_hbm, blk_v, sub_v, rows_v, out_v,
             sem):
    wid = lax.axis_index("s") * NC + lax.axis_index("c")
    pltpu.sync_copy(blk_hbm.at[wid], blk_v)
    pltpu.sync_copy(sub_hbm.at[wid], sub_v)

    def issue(j, buf):
        return pltpu.async_copy(
            emb_hbm.at[blk_v.at[j]], rows_v.at[buf], sem
        )

    pending = issue(0, 0)
    for j in range(NCHUNK):
        if j + 1 < NCHUNK:
            nxt = issue(j + 1, (j + 1) % 2)
        pending.wait()
        buf = j % 2

        def group(g, carry):
            base = g * 16
            slot_s = lax.iota(jnp.int32, 16) + base
            slot_o = slot_s + CPAIRS
            col_s = sub_v[j, pl.ds(base, 16)] * D
            col_o = sub_v[j, pl.ds(CPAIRS + base, 16)] * D
            buf_i = jnp.full((16,), buf, jnp.int32)
            acc = jnp.zeros((16,), jnp.float32)
            for d in range(D):
                s_v = plsc.load_gather(rows_v, [buf_i, slot_s, col_s + d])
                o_v = plsc.load_gather(rows_v, [buf_i, slot_o, col_o + d])
                acc = acc + s_v * o_v
            out_v[pl.ds(j * CPAIRS + base, 16)] = 1.0 / (1.0 + jnp.exp(-acc))
            return carry

        lax.fori_loop(0, CGROUPS, group, 0)
        if j + 1 < NCHUNK:
            pending = nxt
    pltpu.sync_copy(out_v, out_hbm.at[pl.ds(wid * PAIRS_PER_W, PAIRS_PER_W)])


@jax.jit
def _ip(emb2, blk, sub):
    mesh = plsc.VectorSubcoreMesh(core_axis_name="c", subcore_axis_name="s")
    return pl.kernel(
        _ip_body,
        mesh=mesh,
        compiler_params=pltpu.CompilerParams(needs_layout_passes=False),
        out_type=jax.ShapeDtypeStruct((PAIRS,), jnp.float32),
        scratch_types=[
            pltpu.VMEM((NCHUNK, CHUNK), jnp.int32),
            pltpu.VMEM((NCHUNK, CHUNK), jnp.int32),
            pltpu.VMEM((2, CHUNK, 128), jnp.float32),
            pltpu.VMEM((PAIRS_PER_W,), jnp.float32),
            pltpu.SemaphoreType.DMA,
        ],
    )(emb2, blk, sub)


def kernel(emb, batch_ind):
    emb2 = emb.reshape(SROWS, 128)
    idx = batch_ind.astype(jnp.int32)
    idx = idx.reshape(NW, NCHUNK, CPAIRS, 2).transpose(0, 1, 3, 2)
    idx = idx.reshape(NW, NCHUNK, CHUNK)
    return _ip(emb2, idx // RPS, idx % RPS)
